# Initial kernel scaffold; baseline (speedup 1.0000x reference)
import jax
import jax.numpy as jnp
from jax.experimental import pallas as pl


def kernel(x, idx, delta):
    raise NotImplementedError("write your pallas kernel here")


# speedup vs baseline: 1.1418x; 1.1418x over previous
import functools

import jax
import jax.numpy as jnp
from jax import lax
from jax.experimental import pallas as pl
from jax.experimental.pallas import tpu as pltpu
from jax.experimental.pallas import tpu_sc as plsc

BATCH = 4096
DIM = 128


def _build():
    info = plsc.get_sparse_core_info()
    nc, ns, lanes = info.num_cores, info.num_subcores, info.num_lanes
    nw = nc * ns
    bpw = BATCH // nw

    mesh = plsc.VectorSubcoreMesh(core_axis_name="c", subcore_axis_name="s")

    @functools.partial(
        pl.kernel,
        mesh=mesh,
        out_type=jax.ShapeDtypeStruct((BATCH, DIM), jnp.float32),
        scratch_types=[
            pltpu.VMEM((bpw,), jnp.int32),
            pltpu.VMEM((bpw, DIM), jnp.float32),
            pltpu.VMEM((bpw, DIM), jnp.float32),
            pltpu.SemaphoreType.DMA,
        ],
    )
    def sc_kernel(x_hbm, idx_hbm, delta_hbm, out_hbm, idx_v, rows_v, x_v, sem):
        wid = lax.axis_index("s") * nc + lax.axis_index("c")
        base = wid * bpw
        pltpu.sync_copy(idx_hbm.at[pl.ds(base, bpw)], idx_v)
        gather = pltpu.async_copy(delta_hbm.at[idx_v], rows_v, sem)
        pltpu.sync_copy(x_hbm.at[pl.ds(base, bpw)], x_v)
        gather.wait()

        def body(r, carry):
            for c in range(DIM // lanes):
                sl = pl.ds(c * lanes, lanes)
                x_v[r, sl] = x_v[r, sl] + rows_v[r, sl]
            return carry

        lax.fori_loop(0, bpw, body, 0)
        pltpu.sync_copy(x_v, out_hbm.at[pl.ds(base, bpw)])

    return sc_kernel


_sc_kernel = _build()


@jax.jit
def kernel(x, idx, delta):
    return _sc_kernel(x, idx.astype(jnp.int32), delta)


# speedup vs baseline: 1.1756x; 1.0296x over previous
import functools

import jax
import jax.numpy as jnp
from jax import lax
from jax.experimental import pallas as pl
from jax.experimental.pallas import tpu as pltpu
from jax.experimental.pallas import tpu_sc as plsc

BATCH = 4096
DIM = 128


def _build():
    info = plsc.get_sparse_core_info()
    nc, ns, lanes = info.num_cores, info.num_subcores, info.num_lanes
    nw = nc * ns
    bpw = BATCH // nw

    mesh = plsc.VectorSubcoreMesh(core_axis_name="c", subcore_axis_name="s")

    @functools.partial(
        pl.kernel,
        mesh=mesh,
        out_type=jax.ShapeDtypeStruct((BATCH, DIM), jnp.float32),
        scratch_types=[
            pltpu.VMEM((bpw,), jnp.int32),
            pltpu.VMEM((bpw, DIM), jnp.float32),
            pltpu.SemaphoreType.DMA,
        ],
    )
    def sc_kernel(x_hbm, idx_hbm, delta_hbm, out_hbm, idx_v, x_v, sem):
        wid = lax.axis_index("s") * nc + lax.axis_index("c")
        base = wid * bpw
        pltpu.sync_copy(idx_hbm.at[pl.ds(base, bpw)], idx_v)
        pltpu.sync_copy(x_hbm.at[pl.ds(base, bpw)], x_v)
        pltpu.async_copy(delta_hbm.at[idx_v], x_v, sem, add=True).wait()
        pltpu.sync_copy(x_v, out_hbm.at[pl.ds(base, bpw)])

    return sc_kernel


_sc_kernel = _build()


@jax.jit
def kernel(x, idx, delta):
    return _sc_kernel(x, idx.astype(jnp.int32), delta)


# speedup vs baseline: 1.2188x; 1.0367x over previous
import functools

import jax
import jax.numpy as jnp
from jax import lax
from jax.experimental import pallas as pl
from jax.experimental.pallas import tpu as pltpu
from jax.experimental.pallas import tpu_sc as plsc

BATCH = 4096
DIM = 128


def _build():
    info = plsc.get_sparse_core_info()
    nc, ns, lanes = info.num_cores, info.num_subcores, info.num_lanes
    nw = nc * ns
    bpw = BATCH // nw

    mesh = plsc.VectorSubcoreMesh(core_axis_name="c", subcore_axis_name="s")

    @functools.partial(
        pl.kernel,
        mesh=mesh,
        out_type=jax.ShapeDtypeStruct((BATCH, DIM), jnp.float32),
        scratch_types=[
            pltpu.VMEM((bpw // 2,), jnp.int32),
            pltpu.VMEM((bpw // 2,), jnp.int32),
            pltpu.VMEM((bpw // 2, DIM), jnp.float32),
            pltpu.VMEM((bpw // 2, DIM), jnp.float32),
            pltpu.SemaphoreType.DMA,
            pltpu.SemaphoreType.DMA,
            pltpu.SemaphoreType.DMA,
            pltpu.SemaphoreType.DMA,
            pltpu.SemaphoreType.DMA,
            pltpu.SemaphoreType.DMA,
            pltpu.SemaphoreType.DMA,
            pltpu.SemaphoreType.DMA,
        ],
    )
    def sc_kernel(x_hbm, idx_hbm, delta_hbm, out_hbm,
                  idx_v0, idx_v1, x_v0, x_v1,
                  semi0, semi1, semx0, semx1, semg0, semg1, semo0, semo1):
        wid = lax.axis_index("s") * nc + lax.axis_index("c")
        base = wid * bpw
        half = bpw // 2
        ci0 = pltpu.async_copy(idx_hbm.at[pl.ds(base, half)], idx_v0, semi0)
        ci1 = pltpu.async_copy(idx_hbm.at[pl.ds(base + half, half)], idx_v1, semi1)
        cx0 = pltpu.async_copy(x_hbm.at[pl.ds(base, half)], x_v0, semx0)
        cx1 = pltpu.async_copy(x_hbm.at[pl.ds(base + half, half)], x_v1, semx1)
        ci0.wait()
        cx0.wait()
        g0 = pltpu.async_copy(delta_hbm.at[idx_v0], x_v0, semg0, add=True)
        ci1.wait()
        cx1.wait()
        g1 = pltpu.async_copy(delta_hbm.at[idx_v1], x_v1, semg1, add=True)
        g0.wait()
        o0 = pltpu.async_copy(x_v0, out_hbm.at[pl.ds(base, half)], semo0)
        g1.wait()
        o1 = pltpu.async_copy(x_v1, out_hbm.at[pl.ds(base + half, half)], semo1)
        o0.wait()
        o1.wait()

    return sc_kernel


_sc_kernel = _build()


@jax.jit
def kernel(x, idx, delta):
    return _sc_kernel(x, idx.astype(jnp.int32), delta)
